# baseline (speedup 1.0000x reference)
import functools

import jax
import jax.numpy as jnp
from jax import lax
from jax.experimental import pallas as pl
from jax.experimental.pallas import tpu as pltpu
from jax.experimental.pallas import tpu_sc as plsc


def _tc_tail(table, C0, TW):
    V, H = table.shape
    W = H - C0
    R = 2000
    assert V % R == 0 and C0 % 128 == 0 and W <= TW

    def body(tab_ref, out_ref):
        lane = lax.broadcasted_iota(jnp.int32, (R, TW), 1)
        out_ref[...] = jnp.where(lane < W, tab_ref[...], 0.0)

    return pl.pallas_call(
        body,
        grid=(V // R,),
        in_specs=[pl.BlockSpec((R, TW), lambda i: (i, C0 // TW))],
        out_specs=pl.BlockSpec((R, TW), lambda i: (i, 0)),
        out_shape=jax.ShapeDtypeStruct((V, TW), jnp.float32),
        compiler_params=pltpu.CompilerParams(
            dimension_semantics=("arbitrary",)
        ),
    )(table)


def _sc_gather(table, tail, idx):
    V, H = table.shape
    TW = tail.shape[1]
    C0 = 256
    HP = C0 + TW
    N = idx.shape[0]
    info = plsc.get_sparse_core_info()
    NC, NS = info.num_cores, info.num_subcores
    NW = NC * NS
    assert N % NW == 0
    per_w = N // NW
    CH = 128
    sizes = [CH] * (per_w // CH)
    if per_w % CH:
        sizes.append(per_w % CH)
    offs = [0]
    for s in sizes[:-1]:
        offs.append(offs[-1] + s)
    n_ch = len(sizes)

    mesh = plsc.VectorSubcoreMesh(core_axis_name="c", subcore_axis_name="s")

    @functools.partial(
        pl.kernel,
        mesh=mesh,
        out_type=jax.ShapeDtypeStruct((N, HP), jnp.float32),
        scratch_types=[
            pltpu.VMEM((per_w,), jnp.int32),
            pltpu.VMEM((CH, HP), jnp.float32),
            pltpu.VMEM((CH, HP), jnp.float32),
            pltpu.SemaphoreType.DMA,
            pltpu.SemaphoreType.DMA,
        ],
    )
    def gather_kernel(table_hbm, tail_hbm, idx_hbm, out_hbm, idx_v, rows0, rows1,
                      sem0, sem1):
        wid = lax.axis_index("s") * NC + lax.axis_index("c")
        base = wid * per_w
        pltpu.sync_copy(idx_hbm.at[pl.ds(base, per_w)], idx_v)
        bufs = (rows0, rows1)
        sems = (sem0, sem1)

        def start(j):
            n = sizes[j]
            ids = idx_v.at[pl.ds(offs[j], n)]
            buf = bufs[j % 2]
            sem = sems[j % 2]
            cp_a = pltpu.async_copy(
                table_hbm.at[ids, pl.ds(0, C0)],
                buf.at[pl.ds(0, n), pl.ds(0, C0)], sem)
            cp_b = pltpu.async_copy(
                tail_hbm.at[ids],
                buf.at[pl.ds(0, n), pl.ds(C0, TW)], sem)
            return (cp_a, cp_b)

        def drain(j, cps):
            cps[0].wait()
            cps[1].wait()
            pltpu.sync_copy(
                bufs[j % 2].at[pl.ds(0, sizes[j])],
                out_hbm.at[pl.ds(base + offs[j], sizes[j])],
            )

        cp = start(0)
        for j in range(1, n_ch):
            cp_next = start(j)
            drain(j - 1, cp)
            cp = cp_next
        drain(n_ch - 1, cp)

    return gather_kernel(table, tail, idx)


def _gru_bidir_tc(emb, w_ih_t, w_hh_t, b_ih, b_hh):
    T, B, HP = emb.shape
    H = w_hh_t.shape[1]
    H3 = 3 * H

    def body(emb_ref, wih_ref, whh_ref, bih_ref, bhh_ref, out_ref, hid_ref, h_scr):
        i = pl.program_id(0)

        @pl.when((i == 0) | (i == T))
        def _():
            h_scr[...] = jnp.zeros_like(h_scr)

        x = emb_ref[0].astype(jnp.bfloat16)
        h = h_scr[...]
        gi = jnp.dot(x, wih_ref[0], preferred_element_type=jnp.float32) + bih_ref[0, 0]
        gh = jnp.dot(h.astype(jnp.bfloat16), whh_ref[0],
                     preferred_element_type=jnp.float32) + bhh_ref[0, 0]
        i_r, i_z, i_n = gi[:, :H], gi[:, H:2 * H], gi[:, 2 * H:]
        h_r, h_z, h_n = gh[:, :H], gh[:, H:2 * H], gh[:, 2 * H:]
        r = jax.nn.sigmoid(i_r + h_r)
        z = jax.nn.sigmoid(i_z + h_z)
        n = jnp.tanh(i_n + r * h_n)
        h_new = (1.0 - z) * n + z * h
        h_scr[...] = h_new
        out_ref[0, 0] = h_new
        hid_ref[0] = h_new

    t_of = lambda i: jnp.where(i < T, i, 2 * T - 1 - i)
    d_of = lambda i: jnp.where(i < T, 0, 1)

    out, hid = pl.pallas_call(
        body,
        grid=(2 * T,),
        in_specs=[
            pl.BlockSpec((1, B, HP), lambda i: (t_of(i), 0, 0)),
            pl.BlockSpec((1, HP, H3), lambda i: (d_of(i), 0, 0)),
            pl.BlockSpec((1, H, H3), lambda i: (d_of(i), 0, 0)),
            pl.BlockSpec((1, 1, H3), lambda i: (d_of(i), 0, 0)),
            pl.BlockSpec((1, 1, H3), lambda i: (d_of(i), 0, 0)),
        ],
        out_specs=[
            pl.BlockSpec((1, 1, B, H), lambda i: (d_of(i), t_of(i), 0, 0)),
            pl.BlockSpec((1, B, H), lambda i: (d_of(i), 0, 0)),
        ],
        out_shape=[
            jax.ShapeDtypeStruct((2, T, B, H), jnp.float32),
            jax.ShapeDtypeStruct((2, B, H), jnp.float32),
        ],
        scratch_shapes=[pltpu.VMEM((B, H), jnp.float32)],
        compiler_params=pltpu.CompilerParams(
            dimension_semantics=("arbitrary",)
        ),
    )(emb, w_ih_t, w_hh_t, b_ih, b_hh)
    return out, hid


def kernel(input_seqs, input_lengths, table, w_ih_f, w_hh_f, b_ih_f, b_hh_f,
           w_ih_b, w_hh_b, b_ih_b, b_hh_b):
    T, B = input_seqs.shape
    V, H = table.shape
    C0, TW = 256, 128
    HP = C0 + TW
    tail = _tc_tail(table, C0, TW)
    emb = _sc_gather(table, tail, input_seqs.reshape(T * B)).reshape(T, B, HP)
    w_ih_t = jnp.pad(jnp.stack([w_ih_f.T, w_ih_b.T]),
                     ((0, 0), (0, HP - H), (0, 0))).astype(jnp.bfloat16)
    w_hh_t = jnp.stack([w_hh_f.T, w_hh_b.T]).astype(jnp.bfloat16)
    b_ih = jnp.stack([b_ih_f, b_ih_b])[:, None, :]
    b_hh = jnp.stack([b_hh_f, b_hh_b])[:, None, :]
    out2, hid = _gru_bidir_tc(emb, w_ih_t, w_hh_t, b_ih, b_hh)
    return jnp.concatenate([out2[0], out2[1]], axis=-1), hid


# speedup vs baseline: 1.2255x; 1.2255x over previous
import functools

import jax
import jax.numpy as jnp
from jax import lax
from jax.experimental import pallas as pl
from jax.experimental.pallas import tpu as pltpu
from jax.experimental.pallas import tpu_sc as plsc


def _tc_tail(table, C0, TW):
    V, H = table.shape
    W = H - C0
    R = 2000
    assert V % R == 0 and C0 % 128 == 0 and W <= TW

    def body(tab_ref, out_ref):
        lane = lax.broadcasted_iota(jnp.int32, (R, TW), 1)
        out_ref[...] = jnp.where(lane < W, tab_ref[...], 0.0)

    return pl.pallas_call(
        body,
        grid=(V // R,),
        in_specs=[pl.BlockSpec((R, TW), lambda i: (i, C0 // TW))],
        out_specs=pl.BlockSpec((R, TW), lambda i: (i, 0)),
        out_shape=jax.ShapeDtypeStruct((V, TW), jnp.float32),
        compiler_params=pltpu.CompilerParams(
            dimension_semantics=("arbitrary",)
        ),
    )(table)


def _sc_gather(table, tail, idx):
    V, H = table.shape
    TW = tail.shape[1]
    C0 = 256
    HP = C0 + TW
    N = idx.shape[0]
    info = plsc.get_sparse_core_info()
    NC, NS = info.num_cores, info.num_subcores
    NW = NC * NS
    assert N % NW == 0
    per_w = N // NW
    CH = 128
    sizes = [CH] * (per_w // CH)
    if per_w % CH:
        sizes.append(per_w % CH)
    offs = [0]
    for s in sizes[:-1]:
        offs.append(offs[-1] + s)
    n_ch = len(sizes)

    mesh = plsc.VectorSubcoreMesh(core_axis_name="c", subcore_axis_name="s")

    @functools.partial(
        pl.kernel,
        mesh=mesh,
        out_type=jax.ShapeDtypeStruct((N, HP), jnp.float32),
        scratch_types=[
            pltpu.VMEM((per_w,), jnp.int32),
            pltpu.VMEM((CH, HP), jnp.float32),
            pltpu.VMEM((CH, HP), jnp.float32),
            pltpu.SemaphoreType.DMA,
            pltpu.SemaphoreType.DMA,
        ],
    )
    def gather_kernel(table_hbm, tail_hbm, idx_hbm, out_hbm, idx_v, rows0, rows1,
                      sem0, sem1):
        wid = lax.axis_index("s") * NC + lax.axis_index("c")
        base = wid * per_w
        pltpu.sync_copy(idx_hbm.at[pl.ds(base, per_w)], idx_v)
        bufs = (rows0, rows1)
        sems = (sem0, sem1)

        def start(j):
            n = sizes[j]
            ids = idx_v.at[pl.ds(offs[j], n)]
            buf = bufs[j % 2]
            sem = sems[j % 2]
            cp_a = pltpu.async_copy(
                table_hbm.at[ids, pl.ds(0, C0)],
                buf.at[pl.ds(0, n), pl.ds(0, C0)], sem)
            cp_b = pltpu.async_copy(
                tail_hbm.at[ids],
                buf.at[pl.ds(0, n), pl.ds(C0, TW)], sem)
            return (cp_a, cp_b)

        def drain(j, cps):
            cps[0].wait()
            cps[1].wait()
            pltpu.sync_copy(
                bufs[j % 2].at[pl.ds(0, sizes[j])],
                out_hbm.at[pl.ds(base + offs[j], sizes[j])],
            )

        cp = start(0)
        for j in range(1, n_ch):
            cp_next = start(j)
            drain(j - 1, cp)
            cp = cp_next
        drain(n_ch - 1, cp)

    return gather_kernel(table, tail, idx)


def _gru_bidir_tc(emb, w_ih_t, w_hh_t, b_ih, b_hh):
    T, B, HP = emb.shape
    H = w_hh_t.shape[1]
    H3 = 3 * H

    def cell(x, h, wih, whh, bih, bhh):
        gi = jnp.dot(x, wih, preferred_element_type=jnp.float32) + bih
        gh = jnp.dot(h.astype(jnp.bfloat16), whh,
                     preferred_element_type=jnp.float32) + bhh
        i_r, i_z, i_n = gi[:, :H], gi[:, H:2 * H], gi[:, 2 * H:]
        h_r, h_z, h_n = gh[:, :H], gh[:, H:2 * H], gh[:, 2 * H:]
        r = jax.nn.sigmoid(i_r + h_r)
        z = jax.nn.sigmoid(i_z + h_z)
        n = jnp.tanh(i_n + r * h_n)
        return (1.0 - z) * n + z * h

    def body(embf_ref, embb_ref, wih_ref, whh_ref, bih_ref, bhh_ref,
             outf_ref, outb_ref, hid_ref, hf_scr, hb_scr):
        i = pl.program_id(0)

        @pl.when(i == 0)
        def _():
            hf_scr[...] = jnp.zeros_like(hf_scr)
            hb_scr[...] = jnp.zeros_like(hb_scr)

        hf = cell(embf_ref[0].astype(jnp.bfloat16), hf_scr[...],
                  wih_ref[0], whh_ref[0], bih_ref[0, 0], bhh_ref[0, 0])
        hb = cell(embb_ref[0].astype(jnp.bfloat16), hb_scr[...],
                  wih_ref[1], whh_ref[1], bih_ref[1, 0], bhh_ref[1, 0])
        hf_scr[...] = hf
        hb_scr[...] = hb
        outf_ref[0] = hf
        outb_ref[0] = hb
        hid_ref[0] = hf
        hid_ref[1] = hb

    out_f, out_b, hid = pl.pallas_call(
        body,
        grid=(T,),
        in_specs=[
            pl.BlockSpec((1, B, HP), lambda i: (i, 0, 0)),
            pl.BlockSpec((1, B, HP), lambda i: (T - 1 - i, 0, 0)),
            pl.BlockSpec((2, HP, H3), lambda i: (0, 0, 0)),
            pl.BlockSpec((2, H, H3), lambda i: (0, 0, 0)),
            pl.BlockSpec((2, 1, H3), lambda i: (0, 0, 0)),
            pl.BlockSpec((2, 1, H3), lambda i: (0, 0, 0)),
        ],
        out_specs=[
            pl.BlockSpec((1, B, H), lambda i: (i, 0, 0)),
            pl.BlockSpec((1, B, H), lambda i: (T - 1 - i, 0, 0)),
            pl.BlockSpec((2, B, H), lambda i: (0, 0, 0)),
        ],
        out_shape=[
            jax.ShapeDtypeStruct((T, B, H), jnp.float32),
            jax.ShapeDtypeStruct((T, B, H), jnp.float32),
            jax.ShapeDtypeStruct((2, B, H), jnp.float32),
        ],
        scratch_shapes=[pltpu.VMEM((B, H), jnp.float32),
                        pltpu.VMEM((B, H), jnp.float32)],
        compiler_params=pltpu.CompilerParams(
            dimension_semantics=("arbitrary",)
        ),
    )(emb, emb, w_ih_t, w_hh_t, b_ih, b_hh)
    return out_f, out_b, hid


def kernel(input_seqs, input_lengths, table, w_ih_f, w_hh_f, b_ih_f, b_hh_f,
           w_ih_b, w_hh_b, b_ih_b, b_hh_b):
    T, B = input_seqs.shape
    V, H = table.shape
    C0, TW = 256, 128
    HP = C0 + TW
    tail = _tc_tail(table, C0, TW)
    emb = _sc_gather(table, tail, input_seqs.reshape(T * B)).reshape(T, B, HP)
    w_ih_t = jnp.pad(jnp.stack([w_ih_f.T, w_ih_b.T]),
                     ((0, 0), (0, HP - H), (0, 0))).astype(jnp.bfloat16)
    w_hh_t = jnp.stack([w_hh_f.T, w_hh_b.T]).astype(jnp.bfloat16)
    b_ih = jnp.stack([b_ih_f, b_ih_b])[:, None, :]
    b_hh = jnp.stack([b_hh_f, b_hh_b])[:, None, :]
    out_f, out_b, hid = _gru_bidir_tc(emb, w_ih_t, w_hh_t, b_ih, b_hh)
    return jnp.concatenate([out_f, out_b], axis=-1), hid


# speedup vs baseline: 1.2694x; 1.0358x over previous
import functools

import jax
import jax.numpy as jnp
from jax import lax
from jax.experimental import pallas as pl
from jax.experimental.pallas import tpu as pltpu
from jax.experimental.pallas import tpu_sc as plsc


def _tc_tail(table, C0, TW):
    V, H = table.shape
    W = H - C0
    R = 2000
    assert V % R == 0 and C0 % 128 == 0 and W <= TW

    def body(tab_ref, out_ref):
        lane = lax.broadcasted_iota(jnp.int32, (R, TW), 1)
        out_ref[...] = jnp.where(lane < W, tab_ref[...], 0.0)

    return pl.pallas_call(
        body,
        grid=(V // R,),
        in_specs=[pl.BlockSpec((R, TW), lambda i: (i, C0 // TW))],
        out_specs=pl.BlockSpec((R, TW), lambda i: (i, 0)),
        out_shape=jax.ShapeDtypeStruct((V, TW), jnp.float32),
        compiler_params=pltpu.CompilerParams(
            dimension_semantics=("arbitrary",)
        ),
    )(table)


def _sc_gather(table, tail, idx):
    V, H = table.shape
    TW = tail.shape[1]
    C0 = 256
    HP = C0 + TW
    N = idx.shape[0]
    info = plsc.get_sparse_core_info()
    NC, NS = info.num_cores, info.num_subcores
    NW = NC * NS
    assert N % NW == 0
    per_w = N // NW
    CH = 128
    sizes = [CH] * (per_w // CH)
    if per_w % CH:
        sizes.append(per_w % CH)
    offs = [0]
    for s in sizes[:-1]:
        offs.append(offs[-1] + s)
    n_ch = len(sizes)

    mesh = plsc.VectorSubcoreMesh(core_axis_name="c", subcore_axis_name="s")

    @functools.partial(
        pl.kernel,
        mesh=mesh,
        out_type=jax.ShapeDtypeStruct((N, HP), jnp.float32),
        scratch_types=[
            pltpu.VMEM((per_w,), jnp.int32),
            pltpu.VMEM((CH, HP), jnp.float32),
            pltpu.VMEM((CH, HP), jnp.float32),
            pltpu.SemaphoreType.DMA,
            pltpu.SemaphoreType.DMA,
        ],
    )
    def gather_kernel(table_hbm, tail_hbm, idx_hbm, out_hbm, idx_v, rows0, rows1,
                      sem0, sem1):
        wid = lax.axis_index("s") * NC + lax.axis_index("c")
        base = wid * per_w
        pltpu.sync_copy(idx_hbm.at[pl.ds(base, per_w)], idx_v)
        bufs = (rows0, rows1)
        sems = (sem0, sem1)

        def start(j):
            n = sizes[j]
            ids = idx_v.at[pl.ds(offs[j], n)]
            buf = bufs[j % 2]
            sem = sems[j % 2]
            cp_a = pltpu.async_copy(
                table_hbm.at[ids, pl.ds(0, C0)],
                buf.at[pl.ds(0, n), pl.ds(0, C0)], sem)
            cp_b = pltpu.async_copy(
                tail_hbm.at[ids],
                buf.at[pl.ds(0, n), pl.ds(C0, TW)], sem)
            return (cp_a, cp_b)

        def drain(j, cps):
            cps[0].wait()
            cps[1].wait()
            pltpu.sync_copy(
                bufs[j % 2].at[pl.ds(0, sizes[j])],
                out_hbm.at[pl.ds(base + offs[j], sizes[j])],
            )

        cp = start(0)
        for j in range(1, n_ch):
            cp_next = start(j)
            drain(j - 1, cp)
            cp = cp_next
        drain(n_ch - 1, cp)

    return gather_kernel(table, tail, idx)


def _gru_bidir_tc(emb, w_ih_t, w_hh_t, b_ih, b_hh):
    T, B, HP = emb.shape
    H = w_hh_t.shape[1]
    G = w_ih_t.shape[2] // 3
    H3 = 3 * G

    def cell(x, h, wih, whh, bih, bhh):
        gi = jnp.dot(x, wih, preferred_element_type=jnp.float32) + bih
        gh = jnp.dot(h.astype(jnp.bfloat16), whh,
                     preferred_element_type=jnp.float32) + bhh
        i_r, i_z, i_n = gi[:, :H], gi[:, G:G + H], gi[:, 2 * G:2 * G + H]
        h_r, h_z, h_n = gh[:, :H], gh[:, G:G + H], gh[:, 2 * G:2 * G + H]
        r = 0.5 + 0.5 * jnp.tanh(0.5 * (i_r + h_r))
        z = 0.5 + 0.5 * jnp.tanh(0.5 * (i_z + h_z))
        n = jnp.tanh(i_n + r * h_n)
        return n + z * (h - n)

    def body(embf_ref, embb_ref, wih_ref, whh_ref, bih_ref, bhh_ref,
             outf_ref, outb_ref, hid_ref, hf_scr, hb_scr):
        i = pl.program_id(0)

        @pl.when(i == 0)
        def _():
            hf_scr[...] = jnp.zeros_like(hf_scr)
            hb_scr[...] = jnp.zeros_like(hb_scr)

        hf = cell(embf_ref[0].astype(jnp.bfloat16), hf_scr[...],
                  wih_ref[0], whh_ref[0], bih_ref[0, 0], bhh_ref[0, 0])
        hb = cell(embb_ref[0].astype(jnp.bfloat16), hb_scr[...],
                  wih_ref[1], whh_ref[1], bih_ref[1, 0], bhh_ref[1, 0])
        hf_scr[...] = hf
        hb_scr[...] = hb
        outf_ref[0] = hf
        outb_ref[0] = hb
        hid_ref[0] = hf
        hid_ref[1] = hb

    out_f, out_b, hid = pl.pallas_call(
        body,
        grid=(T,),
        in_specs=[
            pl.BlockSpec((1, B, HP), lambda i: (i, 0, 0)),
            pl.BlockSpec((1, B, HP), lambda i: (T - 1 - i, 0, 0)),
            pl.BlockSpec((2, HP, H3), lambda i: (0, 0, 0)),
            pl.BlockSpec((2, H, H3), lambda i: (0, 0, 0)),
            pl.BlockSpec((2, 1, H3), lambda i: (0, 0, 0)),
            pl.BlockSpec((2, 1, H3), lambda i: (0, 0, 0)),
        ],
        out_specs=[
            pl.BlockSpec((1, B, H), lambda i: (i, 0, 0)),
            pl.BlockSpec((1, B, H), lambda i: (T - 1 - i, 0, 0)),
            pl.BlockSpec((2, B, H), lambda i: (0, 0, 0)),
        ],
        out_shape=[
            jax.ShapeDtypeStruct((T, B, H), jnp.float32),
            jax.ShapeDtypeStruct((T, B, H), jnp.float32),
            jax.ShapeDtypeStruct((2, B, H), jnp.float32),
        ],
        scratch_shapes=[pltpu.VMEM((B, H), jnp.float32),
                        pltpu.VMEM((B, H), jnp.float32)],
        compiler_params=pltpu.CompilerParams(
            dimension_semantics=("arbitrary",)
        ),
    )(emb, emb, w_ih_t, w_hh_t, b_ih, b_hh)
    return out_f, out_b, hid


def kernel(input_seqs, input_lengths, table, w_ih_f, w_hh_f, b_ih_f, b_hh_f,
           w_ih_b, w_hh_b, b_ih_b, b_hh_b):
    T, B = input_seqs.shape
    V, H = table.shape
    C0, TW = 256, 128
    HP = C0 + TW
    G = ((H + 127) // 128) * 128
    tail = _tc_tail(table, C0, TW)
    emb = _sc_gather(table, tail, input_seqs.reshape(T * B)).reshape(T, B, HP)

    def prep_w(wf, wb, kp):
        w = jnp.stack([wf.T, wb.T])
        w = w.reshape(2, H, 3, H)
        w = jnp.pad(w, ((0, 0), (0, kp - H), (0, 0), (0, G - H)))
        return w.reshape(2, kp, 3 * G).astype(jnp.bfloat16)

    def prep_b(bf, bb):
        b = jnp.stack([bf, bb]).reshape(2, 3, H)
        b = jnp.pad(b, ((0, 0), (0, 0), (0, G - H)))
        return b.reshape(2, 1, 3 * G)

    w_ih_t = prep_w(w_ih_f, w_ih_b, HP)
    w_hh_t = prep_w(w_hh_f, w_hh_b, H)
    b_ih = prep_b(b_ih_f, b_ih_b)
    b_hh = prep_b(b_hh_f, b_hh_b)
    out_f, out_b, hid = _gru_bidir_tc(emb, w_ih_t, w_hh_t, b_ih, b_hh)
    return jnp.concatenate([out_f, out_b], axis=-1), hid


# speedup vs baseline: 1.3079x; 1.0303x over previous
import functools

import jax
import jax.numpy as jnp
from jax import lax
from jax.experimental import pallas as pl
from jax.experimental.pallas import tpu as pltpu
from jax.experimental.pallas import tpu_sc as plsc


def _tc_tail(table, C0, TW):
    V, H = table.shape
    W = H - C0
    R = 2000
    assert V % R == 0 and C0 % 128 == 0 and W <= TW

    def body(tab_ref, out_ref):
        lane = lax.broadcasted_iota(jnp.int32, (R, TW), 1)
        vals = jnp.where(lane < W, tab_ref[...], 0.0)
        out_ref[...] = jnp.where(lane == W, 1.0, vals)

    return pl.pallas_call(
        body,
        grid=(V // R,),
        in_specs=[pl.BlockSpec((R, TW), lambda i: (i, C0 // TW))],
        out_specs=pl.BlockSpec((R, TW), lambda i: (i, 0)),
        out_shape=jax.ShapeDtypeStruct((V, TW), jnp.float32),
        compiler_params=pltpu.CompilerParams(
            dimension_semantics=("arbitrary",)
        ),
    )(table)


def _sc_gather(table, tail, idx):
    V, H = table.shape
    TW = tail.shape[1]
    C0 = 256
    HP = C0 + TW
    N = idx.shape[0]
    info = plsc.get_sparse_core_info()
    NC, NS = info.num_cores, info.num_subcores
    NW = NC * NS
    assert N % NW == 0
    per_w = N // NW
    CH = 128
    sizes = [CH] * (per_w // CH)
    if per_w % CH:
        sizes.append(per_w % CH)
    offs = [0]
    for s in sizes[:-1]:
        offs.append(offs[-1] + s)
    n_ch = len(sizes)

    mesh = plsc.VectorSubcoreMesh(core_axis_name="c", subcore_axis_name="s")

    @functools.partial(
        pl.kernel,
        mesh=mesh,
        out_type=jax.ShapeDtypeStruct((N, HP), jnp.float32),
        scratch_types=[
            pltpu.VMEM((per_w,), jnp.int32),
            pltpu.VMEM((CH, HP), jnp.float32),
            pltpu.VMEM((CH, HP), jnp.float32),
            pltpu.SemaphoreType.DMA,
            pltpu.SemaphoreType.DMA,
        ],
    )
    def gather_kernel(table_hbm, tail_hbm, idx_hbm, out_hbm, idx_v, rows0, rows1,
                      sem0, sem1):
        wid = lax.axis_index("s") * NC + lax.axis_index("c")
        base = wid * per_w
        pltpu.sync_copy(idx_hbm.at[pl.ds(base, per_w)], idx_v)
        bufs = (rows0, rows1)
        sems = (sem0, sem1)

        def start(j):
            n = sizes[j]
            ids = idx_v.at[pl.ds(offs[j], n)]
            buf = bufs[j % 2]
            sem = sems[j % 2]
            cp_a = pltpu.async_copy(
                table_hbm.at[ids, pl.ds(0, C0)],
                buf.at[pl.ds(0, n), pl.ds(0, C0)], sem)
            cp_b = pltpu.async_copy(
                tail_hbm.at[ids],
                buf.at[pl.ds(0, n), pl.ds(C0, TW)], sem)
            return (cp_a, cp_b)

        def drain(j, cps):
            cps[0].wait()
            cps[1].wait()
            pltpu.sync_copy(
                bufs[j % 2].at[pl.ds(0, sizes[j])],
                out_hbm.at[pl.ds(base + offs[j], sizes[j])],
            )

        cp = start(0)
        for j in range(1, n_ch):
            cp_next = start(j)
            drain(j - 1, cp)
            cp = cp_next
        drain(n_ch - 1, cp)

    return gather_kernel(table, tail, idx)


def _gru_bidir_tc(emb, w_ih_t, w_hh_t, H):
    T, B, HP = emb.shape
    G = w_ih_t.shape[2] // 3
    H3 = 3 * G

    def cell(x, h_scr, wih, whh):
        hp = h_scr[...]
        gi = jnp.dot(x, wih, preferred_element_type=jnp.float32)
        gh = jnp.dot(hp.astype(jnp.bfloat16), whh,
                     preferred_element_type=jnp.float32)
        i_r, i_z, i_n = gi[:, :H], gi[:, G:G + H], gi[:, 2 * G:2 * G + H]
        h_r, h_z, h_n = gh[:, :H], gh[:, G:G + H], gh[:, 2 * G:2 * G + H]
        r = 0.5 + 0.5 * jnp.tanh(i_r + h_r)
        z = 0.5 + 0.5 * jnp.tanh(i_z + h_z)
        n = jnp.tanh(i_n + r * h_n)
        return n + z * (hp[:, :H] - n)

    def body(embf_ref, embb_ref, wih_ref, whh_ref,
             outf_ref, outb_ref, hid_ref, hf_scr, hb_scr):
        i = pl.program_id(0)

        @pl.when(i == 0)
        def _():
            lane = lax.broadcasted_iota(jnp.int32, (B, HP), 1)
            init = jnp.where(lane == H, 1.0, 0.0)
            hf_scr[...] = init
            hb_scr[...] = init

        hf = cell(embf_ref[0].astype(jnp.bfloat16), hf_scr, wih_ref[0], whh_ref[0])
        hb = cell(embb_ref[0].astype(jnp.bfloat16), hb_scr, wih_ref[1], whh_ref[1])
        hf_scr[:, :H] = hf
        hb_scr[:, :H] = hb
        outf_ref[0] = hf
        outb_ref[0] = hb

        @pl.when(i == T - 1)
        def _():
            hid_ref[0] = hf
            hid_ref[1] = hb

    out_f, out_b, hid = pl.pallas_call(
        body,
        grid=(T,),
        in_specs=[
            pl.BlockSpec((1, B, HP), lambda i: (i, 0, 0)),
            pl.BlockSpec((1, B, HP), lambda i: (T - 1 - i, 0, 0)),
            pl.BlockSpec((2, HP, H3), lambda i: (0, 0, 0)),
            pl.BlockSpec((2, HP, H3), lambda i: (0, 0, 0)),
        ],
        out_specs=[
            pl.BlockSpec((1, B, H), lambda i: (i, 0, 0)),
            pl.BlockSpec((1, B, H), lambda i: (T - 1 - i, 0, 0)),
            pl.BlockSpec((2, B, H), lambda i: (0, 0, 0)),
        ],
        out_shape=[
            jax.ShapeDtypeStruct((T, B, H), jnp.float32),
            jax.ShapeDtypeStruct((T, B, H), jnp.float32),
            jax.ShapeDtypeStruct((2, B, H), jnp.float32),
        ],
        scratch_shapes=[pltpu.VMEM((B, HP), jnp.float32),
                        pltpu.VMEM((B, HP), jnp.float32)],
        compiler_params=pltpu.CompilerParams(
            dimension_semantics=("arbitrary",)
        ),
    )(emb, emb, w_ih_t, w_hh_t)
    return out_f, out_b, hid


def kernel(input_seqs, input_lengths, table, w_ih_f, w_hh_f, b_ih_f, b_hh_f,
           w_ih_b, w_hh_b, b_ih_b, b_hh_b):
    T, B = input_seqs.shape
    V, H = table.shape
    C0, TW = 256, 128
    HP = C0 + TW
    G = ((H + 127) // 128) * 128
    tail = _tc_tail(table, C0, TW)
    emb = _sc_gather(table, tail, input_seqs.reshape(T * B)).reshape(T, B, HP)

    def prep_w(wf, wb, brow):
        w = jnp.stack([wf.T, wb.T]).reshape(2, H, 3, H)
        w = jnp.concatenate([w, brow[:, None]], axis=1)
        w = jnp.pad(w, ((0, 0), (0, HP - H - 1), (0, 0), (0, G - H)))
        w = w * jnp.array([0.5, 0.5, 1.0]).reshape(1, 1, 3, 1)
        return w.reshape(2, HP, 3 * G).astype(jnp.bfloat16)

    bih3 = jnp.stack([b_ih_f, b_ih_b]).reshape(2, 3, H)
    bhh3 = jnp.stack([b_hh_f, b_hh_b]).reshape(2, 3, H)
    rz = jnp.array([1.0, 1.0, 0.0]).reshape(1, 3, 1)
    w_ih_t = prep_w(w_ih_f, w_ih_b, bih3 + bhh3 * rz)
    w_hh_t = prep_w(w_hh_f, w_hh_b, bhh3 * (1.0 - rz))
    out_f, out_b, hid = _gru_bidir_tc(emb, w_ih_t, w_hh_t, H)
    return jnp.concatenate([out_f, out_b], axis=-1), hid


# speedup vs baseline: 1.3604x; 1.0401x over previous
import functools

import jax
import jax.numpy as jnp
from jax import lax
from jax.experimental import pallas as pl
from jax.experimental.pallas import tpu as pltpu
from jax.experimental.pallas import tpu_sc as plsc


def _tc_tail(table, C0, TW):
    V, H = table.shape
    W = H - C0
    R = 2000
    assert V % R == 0 and C0 % 128 == 0 and W <= TW

    def body(tab_ref, out_ref):
        lane = lax.broadcasted_iota(jnp.int32, (R, TW), 1)
        vals = jnp.where(lane < W, tab_ref[...], 0.0)
        out_ref[...] = jnp.where(lane == W, 1.0, vals)

    return pl.pallas_call(
        body,
        grid=(V // R,),
        in_specs=[pl.BlockSpec((R, TW), lambda i: (i, C0 // TW))],
        out_specs=pl.BlockSpec((R, TW), lambda i: (i, 0)),
        out_shape=jax.ShapeDtypeStruct((V, TW), jnp.float32),
        compiler_params=pltpu.CompilerParams(
            dimension_semantics=("arbitrary",)
        ),
    )(table)


def _sc_gather(src, idx, C0):
    V, H = src.shape
    N = idx.shape[0]
    info = plsc.get_sparse_core_info()
    NC, NS = info.num_cores, info.num_subcores
    NW = NC * NS
    assert N % NW == 0 and C0 % 128 == 0
    per_w = N // NW
    CH = 128
    sizes = [CH] * (per_w // CH)
    if per_w % CH:
        sizes.append(per_w % CH)
    offs = [0]
    for s in sizes[:-1]:
        offs.append(offs[-1] + s)
    n_ch = len(sizes)

    mesh = plsc.VectorSubcoreMesh(core_axis_name="c", subcore_axis_name="s")

    @functools.partial(
        pl.kernel,
        mesh=mesh,
        out_type=jax.ShapeDtypeStruct((N, C0), jnp.float32),
        scratch_types=[
            pltpu.VMEM((per_w,), jnp.int32),
            pltpu.VMEM((CH, C0), jnp.float32),
            pltpu.VMEM((CH, C0), jnp.float32),
            pltpu.SemaphoreType.DMA,
            pltpu.SemaphoreType.DMA,
        ],
    )
    def gather_kernel(src_hbm, idx_hbm, out_hbm, idx_v, rows0, rows1,
                      sem0, sem1):
        wid = lax.axis_index("s") * NC + lax.axis_index("c")
        base = wid * per_w
        pltpu.sync_copy(idx_hbm.at[pl.ds(base, per_w)], idx_v)
        bufs = (rows0, rows1)
        sems = (sem0, sem1)

        def start(j):
            n = sizes[j]
            ids = idx_v.at[pl.ds(offs[j], n)]
            buf = bufs[j % 2].at[pl.ds(0, n)]
            if C0 < H:
                src_slc = src_hbm.at[ids, pl.ds(0, C0)]
            else:
                src_slc = src_hbm.at[ids]
            return pltpu.async_copy(src_slc, buf, sems[j % 2])

        def drain(j, cp):
            cp.wait()
            pltpu.sync_copy(
                bufs[j % 2].at[pl.ds(0, sizes[j])],
                out_hbm.at[pl.ds(base + offs[j], sizes[j])],
            )

        cp = start(0)
        for j in range(1, n_ch):
            cp_next = start(j)
            drain(j - 1, cp)
            cp = cp_next
        drain(n_ch - 1, cp)

    return gather_kernel(src, idx)


def _gru_bidir_tc(emb_a, emb_b, w_ia, w_tn, w_cat, H):
    T, B, C0 = emb_a.shape
    TW = emb_b.shape[2]
    KC = w_cat.shape[1]
    G = w_ia.shape[2] // 3
    H3 = 3 * G

    def cell(xa, xb, hx_scr, wia, wtn, wcat):
        hx = hx_scr[...]
        gi = jnp.dot(xa, wia, preferred_element_type=jnp.float32)
        gt = jnp.dot(xb, wtn, preferred_element_type=jnp.float32)
        gh = jnp.dot(hx.astype(jnp.bfloat16), wcat,
                     preferred_element_type=jnp.float32)
        i_r, i_z, i_n = gi[:, :H], gi[:, G:G + H], gi[:, 2 * G:2 * G + H]
        h_r, h_z, h_n = gh[:, :H], gh[:, G:G + H], gh[:, 2 * G:2 * G + H]
        r = 0.5 + 0.5 * jnp.tanh(i_r + h_r)
        z = 0.5 + 0.5 * jnp.tanh(i_z + h_z)
        n = jnp.tanh(i_n + gt[:, :H] + r * h_n)
        return n + z * (hx[:, TW:TW + H] - n)

    def body(embfa_ref, embfb_ref, embba_ref, embbb_ref, wia_ref, wtn_ref,
             wcat_ref, outf_ref, outb_ref, hid_ref, hxf_scr, hxb_scr):
        i = pl.program_id(0)

        @pl.when(i == 0)
        def _():
            lane = lax.broadcasted_iota(jnp.int32, (B, KC), 1)
            init = jnp.where(lane == TW + H, 1.0, 0.0)
            hxf_scr[...] = init
            hxb_scr[...] = init

        hxf_scr[:, :TW] = embfb_ref[0]
        hxb_scr[:, :TW] = embbb_ref[0]
        hf = cell(embfa_ref[0].astype(jnp.bfloat16),
                  embfb_ref[0].astype(jnp.bfloat16), hxf_scr,
                  wia_ref[0], wtn_ref[0], wcat_ref[0])
        hb = cell(embba_ref[0].astype(jnp.bfloat16),
                  embbb_ref[0].astype(jnp.bfloat16), hxb_scr,
                  wia_ref[1], wtn_ref[1], wcat_ref[1])
        hxf_scr[:, TW:TW + H] = hf
        hxb_scr[:, TW:TW + H] = hb
        outf_ref[0] = hf
        outb_ref[0] = hb

        @pl.when(i == T - 1)
        def _():
            hid_ref[0] = hf
            hid_ref[1] = hb

    out_f, out_b, hid = pl.pallas_call(
        body,
        grid=(T,),
        in_specs=[
            pl.BlockSpec((1, B, C0), lambda i: (i, 0, 0)),
            pl.BlockSpec((1, B, TW), lambda i: (i, 0, 0)),
            pl.BlockSpec((1, B, C0), lambda i: (T - 1 - i, 0, 0)),
            pl.BlockSpec((1, B, TW), lambda i: (T - 1 - i, 0, 0)),
            pl.BlockSpec((2, C0, H3), lambda i: (0, 0, 0)),
            pl.BlockSpec((2, TW, G), lambda i: (0, 0, 0)),
            pl.BlockSpec((2, KC, H3), lambda i: (0, 0, 0)),
        ],
        out_specs=[
            pl.BlockSpec((1, B, H), lambda i: (i, 0, 0)),
            pl.BlockSpec((1, B, H), lambda i: (T - 1 - i, 0, 0)),
            pl.BlockSpec((2, B, H), lambda i: (0, 0, 0)),
        ],
        out_shape=[
            jax.ShapeDtypeStruct((T, B, H), jnp.float32),
            jax.ShapeDtypeStruct((T, B, H), jnp.float32),
            jax.ShapeDtypeStruct((2, B, H), jnp.float32),
        ],
        scratch_shapes=[pltpu.VMEM((B, KC), jnp.float32),
                        pltpu.VMEM((B, KC), jnp.float32)],
        compiler_params=pltpu.CompilerParams(
            dimension_semantics=("arbitrary",)
        ),
    )(emb_a, emb_b, emb_a, emb_b, w_ia, w_tn, w_cat)
    return out_f, out_b, hid


def kernel(input_seqs, input_lengths, table, w_ih_f, w_hh_f, b_ih_f, b_hh_f,
           w_ih_b, w_hh_b, b_ih_b, b_hh_b):
    T, B = input_seqs.shape
    V, H = table.shape
    C0, TW = 256, 128
    G = ((H + 127) // 128) * 128
    KH = G
    idx = input_seqs.reshape(T * B)
    tail = _tc_tail(table, C0, TW)
    emb_a = _sc_gather(table, idx, C0).reshape(T, B, C0)
    emb_b = _sc_gather(tail, idx, TW).reshape(T, B, TW)

    bih3 = jnp.stack([b_ih_f, b_ih_b]).reshape(2, 3, H)
    bhh3 = jnp.stack([b_hh_f, b_hh_b]).reshape(2, 3, H)
    rz = jnp.array([1.0, 1.0, 0.0]).reshape(1, 3, 1)
    scale = jnp.array([0.5, 0.5, 1.0]).reshape(1, 1, 3, 1)

    def pack(w4):
        w4 = jnp.pad(w4, ((0, 0), (0, 0), (0, 0), (0, G - H))) * scale
        return w4.reshape(2, w4.shape[1], 3 * G).astype(jnp.bfloat16)

    wih4 = jnp.stack([w_ih_f.T, w_ih_b.T]).reshape(2, H, 3, H)
    whh4 = jnp.stack([w_hh_f.T, w_hh_b.T]).reshape(2, H, 3, H)
    rz4 = rz[:, None]
    wcat4 = jnp.concatenate([
        wih4[:, C0:H] * rz4,
        ((bih3 + bhh3) * rz)[:, None],
        jnp.zeros((2, TW - (H - C0) - 1, 3, H)),
        whh4,
        (bhh3 * (1.0 - rz))[:, None],
        jnp.zeros((2, KH - H - 1, 3, H)),
    ], axis=1)
    wtn3 = jnp.concatenate([
        wih4[:, C0:H, 2],
        bih3[:, 2][:, None],
        jnp.zeros((2, TW - (H - C0) - 1, H)),
    ], axis=1)
    w_ia = pack(wih4[:, :C0])
    w_cat = pack(wcat4)
    w_tn = jnp.pad(wtn3, ((0, 0), (0, 0), (0, G - H))).astype(jnp.bfloat16)
    out_f, out_b, hid = _gru_bidir_tc(emb_a, emb_b, w_ia, w_tn, w_cat, H)
    return jnp.concatenate([out_f, out_b], axis=-1), hid
